# Initial kernel scaffold; baseline (speedup 1.0000x reference)
import jax
import jax.numpy as jnp
from jax.experimental import pallas as pl


def kernel(feat, feat_a, edge_index, graph_neigh, W1, att_src, att_dst, W2, Wd1, bd1, bn_gamma, bn_beta, Wd2, bd2, Wb, bb):
    raise NotImplementedError("write your pallas kernel here")


# speedup vs baseline: 1.2839x; 1.2839x over previous
import functools

import jax
import jax.numpy as jnp
from jax.experimental import pallas as pl
from jax.experimental.pallas import tpu as pltpu

N = 10000
E = 160000
DIN = 512
DH = 256
DOUT = 64


def _mm1_body(x_ref, w_ref, a_ref, h_ref, e_ref):
    h = jnp.dot(x_ref[...], w_ref[...], preferred_element_type=jnp.float32)
    h_ref[...] = h
    e_ref[...] = jnp.dot(h, a_ref[...], preferred_element_type=jnp.float32)


def _mm1(x2, W1, a_pad):
    BM = 1000
    grid = (x2.shape[0] // BM,)
    return pl.pallas_call(
        _mm1_body,
        grid=grid,
        in_specs=[
            pl.BlockSpec((BM, DIN), lambda i: (i, 0)),
            pl.BlockSpec((DIN, DH), lambda i: (0, 0)),
            pl.BlockSpec((DH, 128), lambda i: (0, 0)),
        ],
        out_specs=[
            pl.BlockSpec((BM, DH), lambda i: (i, 0)),
            pl.BlockSpec((BM, 128), lambda i: (i, 0)),
        ],
        out_shape=[
            jax.ShapeDtypeStruct((x2.shape[0], DH), jnp.float32),
            jax.ShapeDtypeStruct((x2.shape[0], 128), jnp.float32),
        ],
    )(x2, W1, a_pad)


def _readout_body(gn_ref, rhs_ref, p_ref, rs_ref):
    gn = gn_ref[...]
    p_ref[...] = jnp.dot(gn, rhs_ref[...], preferred_element_type=jnp.float32)
    rs_ref[...] = jnp.broadcast_to(
        jnp.sum(gn, axis=1, keepdims=True), rs_ref.shape
    )


def _readout_mm(gn, rhs):
    BM = 200
    grid = (N // BM,)
    return pl.pallas_call(
        _readout_body,
        grid=grid,
        in_specs=[
            pl.BlockSpec((BM, N), lambda i: (i, 0)),
            pl.BlockSpec((N, 128), lambda i: (0, 0)),
        ],
        out_specs=[
            pl.BlockSpec((BM, 128), lambda i: (i, 0)),
            pl.BlockSpec((BM, 128), lambda i: (i, 0)),
        ],
        out_shape=[
            jax.ShapeDtypeStruct((N, 128), jnp.float32),
            jax.ShapeDtypeStruct((N, 128), jnp.float32),
        ],
    )(gn, rhs)


def _edge_attn(h, es, ed, src, dst):
    ex = jnp.exp(jax.nn.sigmoid(es[src] + ed[dst]))
    s = jax.ops.segment_sum(ex, dst, num_segments=N)
    alpha = ex / s[dst]
    return jax.ops.segment_sum(h[src] * alpha[:, None], dst, num_segments=N)


def kernel(feat, feat_a, edge_index, graph_neigh, W1, att_src, att_dst, W2,
           Wd1, bd1, bn_gamma, bn_beta, Wd2, bd2, Wb, bb):
    src = edge_index[0]
    dst = edge_index[1]

    x2 = jnp.concatenate([feat, feat_a], axis=0)
    a_pad = jnp.zeros((DH, 128), jnp.float32)
    a_pad = a_pad.at[:, 0].set(att_src).at[:, 1].set(att_dst)
    H, ESED = _mm1(x2, W1, a_pad)
    h = H[:N]
    ha = H[N:]
    es, ed = ESED[:N, 0], ESED[:N, 1]
    esa, eda = ESED[N:, 0], ESED[N:, 1]

    o = _edge_attn(h, es, ed, src, dst)
    oa = _edge_attn(ha, esa, eda, src, dst)

    h1 = jax.nn.elu(o)
    h1a = jax.nn.elu(oa)
    h2 = h1 @ W2
    h2a = h1a @ W2

    d = h2 @ Wd1 + bd1
    mu = jnp.mean(d, axis=0)
    var = jnp.var(d, axis=0)
    d = (d - mu) / jnp.sqrt(var + 1e-5) * bn_gamma + bn_beta
    d = jax.nn.elu(d)
    h3 = d @ Wd2 + bd2

    rhs = jnp.concatenate(
        [jnp.pad(h2, ((0, 0), (0, 0))), h2a], axis=1)
    P, RS = _readout_mm(graph_neigh, rhs)
    rs = RS[:, :1]
    gsum = P[:, :DOUT] / rs
    gasum = P[:, DOUT:] / rs
    gn_ = gsum / jnp.maximum(jnp.linalg.norm(gsum, axis=1, keepdims=True), 1e-12)
    gan_ = gasum / jnp.maximum(jnp.linalg.norm(gasum, axis=1, keepdims=True), 1e-12)
    g = jax.nn.sigmoid(gn_)
    ga = jax.nn.sigmoid(gan_)

    cW = g @ Wb.T
    caW = ga @ Wb.T
    ret = jnp.concatenate([
        jnp.sum(h2 * cW, axis=1, keepdims=True) + bb,
        jnp.sum(h2a * cW, axis=1, keepdims=True) + bb,
    ], axis=1)
    ret_a = jnp.concatenate([
        jnp.sum(h2a * caW, axis=1, keepdims=True) + bb,
        jnp.sum(h2 * caW, axis=1, keepdims=True) + bb,
    ], axis=1)
    return (h2, h3, ret, ret_a, h2, h2a)


# speedup vs baseline: 1.4498x; 1.1293x over previous
import functools

import jax
import jax.numpy as jnp
from jax import lax
from jax.experimental import pallas as pl
from jax.experimental.pallas import tpu as pltpu
from jax.experimental.pallas import tpu_sc as plsc

N = 10000
E = 160000
DIN = 512
DH = 256
DOUT = 64


def _mm1_body(x_ref, w_ref, a_ref, h_ref, e_ref):
    h = jnp.dot(x_ref[...], w_ref[...], preferred_element_type=jnp.float32)
    h_ref[...] = h
    e_ref[...] = jnp.dot(h, a_ref[...], preferred_element_type=jnp.float32)


def _mm1(x2, W1, a_pad):
    BM = 1000
    grid = (x2.shape[0] // BM,)
    return pl.pallas_call(
        _mm1_body,
        grid=grid,
        in_specs=[
            pl.BlockSpec((BM, DIN), lambda i: (i, 0)),
            pl.BlockSpec((DIN, DH), lambda i: (0, 0)),
            pl.BlockSpec((DH, 128), lambda i: (0, 0)),
        ],
        out_specs=[
            pl.BlockSpec((BM, DH), lambda i: (i, 0)),
            pl.BlockSpec((BM, 128), lambda i: (i, 0)),
        ],
        out_shape=[
            jax.ShapeDtypeStruct((x2.shape[0], DH), jnp.float32),
            jax.ShapeDtypeStruct((x2.shape[0], 128), jnp.float32),
        ],
    )(x2, W1, a_pad)


def _readout_body(gn_ref, rhs_ref, p_ref, rs_ref):
    gn = gn_ref[...]
    p_ref[...] = jnp.dot(gn, rhs_ref[...], preferred_element_type=jnp.float32)
    rs_ref[...] = jnp.broadcast_to(
        jnp.sum(gn, axis=1, keepdims=True), rs_ref.shape
    )


def _readout_mm(gn, rhs):
    BM = 200
    grid = (N // BM,)
    return pl.pallas_call(
        _readout_body,
        grid=grid,
        in_specs=[
            pl.BlockSpec((BM, N), lambda i: (i, 0)),
            pl.BlockSpec((N, 128), lambda i: (0, 0)),
        ],
        out_specs=[
            pl.BlockSpec((BM, 128), lambda i: (i, 0)),
            pl.BlockSpec((BM, 128), lambda i: (i, 0)),
        ],
        out_shape=[
            jax.ShapeDtypeStruct((N, 128), jnp.float32),
            jax.ShapeDtypeStruct((N, 128), jnp.float32),
        ],
    )(gn, rhs)


NC = 2
NS = 16
NW = NC * NS
NPW = 320
NPAD = NW * NPW
CHUNK = 1600
NCHUNK = E // CHUNK
VPC = CHUNK // 16


def _sigexp(es_v, ed_v):
    e = es_v + ed_v
    sig = 1.0 / (1.0 + jnp.exp(-e))
    return jnp.exp(sig)


def _edge_sc_body(src_hbm, dst_hbm, es_hbm, ed_hbm, h_hbm,
                  out0_hbm, out1_hbm,
                  es_v, ed_v, srcb, dstb, s_v, rinv_v,
                  bsrc, bloc, balp, rows_v, out_v, sem):
    wid = lax.axis_index("s") * NC + lax.axis_index("c")
    lo = wid * NPW
    iota = lax.iota(jnp.int32, 16)
    outs = (out0_hbm, out1_hbm)
    z16f = jnp.zeros((16,), jnp.float32)
    z16i = jnp.zeros((16,), jnp.int32)

    def zb(i, _):
        bsrc[pl.ds(i * 16, 16)] = z16i
        return 0

    lax.fori_loop(0, (CHUNK + 16) // 16, zb, 0)

    for f in (0, 1):
        pltpu.sync_copy(es_hbm.at[pl.ds(f * N, N)], es_v)
        pltpu.sync_copy(ed_hbm.at[pl.ds(f * N, N)], ed_v)

        def zs(i, _):
            s_v[pl.ds(i * 16, 16)] = z16f
            return 0

        lax.fori_loop(0, NPW // 16, zs, 0)

        def zo(k, _):
            out_v[k // 16, pl.ds((k % 16) * 16, 16)] = z16f
            return 0

        lax.fori_loop(0, NPW * DH // 16, zo, 0)

        def chunk_a(c, _):
            pltpu.sync_copy(src_hbm.at[pl.ds(c * CHUNK, CHUNK)], srcb)
            pltpu.sync_copy(dst_hbm.at[pl.ds(c * CHUNK, CHUNK)], dstb)

            def vec_a(i, _):
                sv = srcb[pl.ds(i * 16, 16)]
                dv = dstb[pl.ds(i * 16, 16)]
                m = (dv >= lo) & (dv < lo + NPW)
                ex = _sigexp(plsc.load_gather(es_v, [sv]),
                             plsc.load_gather(ed_v, [dv]))
                loc = jnp.where(m, dv - lo, 0)
                plsc.addupdate_scatter(s_v, [loc], ex, mask=m)
                return 0

            return lax.fori_loop(0, VPC, vec_a, 0)

        lax.fori_loop(0, NCHUNK, chunk_a, 0)

        def rv(i, _):
            s = s_v[pl.ds(i * 16, 16)]
            rinv_v[pl.ds(i * 16, 16)] = 1.0 / (s + 1e-16)
            return 0

        lax.fori_loop(0, NPW // 16, rv, 0)

        def chunk_b(c, _):
            pltpu.sync_copy(src_hbm.at[pl.ds(c * CHUNK, CHUNK)], srcb)
            pltpu.sync_copy(dst_hbm.at[pl.ds(c * CHUNK, CHUNK)], dstb)

            def vec_b(i, ptr):
                sv = srcb[pl.ds(i * 16, 16)]
                dv = dstb[pl.ds(i * 16, 16)]
                m = (dv >= lo) & (dv < lo + NPW)
                ex = _sigexp(plsc.load_gather(es_v, [sv]),
                             plsc.load_gather(ed_v, [dv]))
                loc = jnp.where(m, dv - lo, 0)
                alpha = ex * plsc.load_gather(rinv_v, [loc])
                plsc.store_compressed(bsrc.at[pl.ds(ptr, 16)], sv + f * N, mask=m)
                plsc.store_compressed(bloc.at[pl.ds(ptr, 16)], loc, mask=m)
                plsc.store_compressed(balp.at[pl.ds(ptr, 16)], alpha, mask=m)
                return ptr + jnp.sum(m.astype(jnp.int32))

            ptr = lax.fori_loop(0, VPC, vec_b, jnp.int32(0))

            def drain(bb, _):
                pltpu.async_copy(
                    h_hbm.at[bsrc.at[pl.ds(bb * 16, 16)]], rows_v, sem
                ).wait()
                lv = bloc[pl.ds(bb * 16, 16)]
                av = balp[pl.ds(bb * 16, 16)]
                valid = (bb * 16 + iota) < ptr

                def fgrp(j, _):
                    for u in range(16):
                        fpv = jnp.broadcast_to(j * 16 + u, (16,)).astype(jnp.int32)
                        col = plsc.load_gather(rows_v, [iota, fpv])
                        plsc.addupdate_scatter(
                            out_v, [lv, fpv], av * col, mask=valid)
                    return 0

                lax.fori_loop(0, DH // 16, fgrp, 0)
                return 0

            lax.fori_loop(0, (ptr + 15) // 16, drain, 0)
            return 0

        lax.fori_loop(0, NCHUNK, chunk_b, 0)
        pltpu.sync_copy(out_v, outs[f].at[pl.ds(lo, NPW)])


def _edge_attn_sc(H, es_all, ed_all, src, dst):
    mesh = plsc.VectorSubcoreMesh(core_axis_name="c", subcore_axis_name="s")
    f = pl.kernel(
        _edge_sc_body,
        out_type=[
            jax.ShapeDtypeStruct((NPAD, DH), jnp.float32),
            jax.ShapeDtypeStruct((NPAD, DH), jnp.float32),
        ],
        mesh=mesh,
        compiler_params=pltpu.CompilerParams(
            use_tc_tiling_on_sc=False, needs_layout_passes=False),
        scratch_types=[
            pltpu.VMEM((N,), jnp.float32),
            pltpu.VMEM((N,), jnp.float32),
            pltpu.VMEM((CHUNK,), jnp.int32),
            pltpu.VMEM((CHUNK,), jnp.int32),
            pltpu.VMEM((NPW,), jnp.float32),
            pltpu.VMEM((NPW,), jnp.float32),
            pltpu.VMEM((CHUNK + 16,), jnp.int32),
            pltpu.VMEM((CHUNK + 16,), jnp.int32),
            pltpu.VMEM((CHUNK + 16,), jnp.float32),
            pltpu.VMEM((16, DH), jnp.float32),
            pltpu.VMEM((NPW, DH), jnp.float32),
            pltpu.SemaphoreType.DMA,
        ],
    )
    o0, o1 = f(src, dst, es_all, ed_all, H)
    return o0[:N], o1[:N]


def kernel(feat, feat_a, edge_index, graph_neigh, W1, att_src, att_dst, W2,
           Wd1, bd1, bn_gamma, bn_beta, Wd2, bd2, Wb, bb):
    src = edge_index[0]
    dst = edge_index[1]

    x2 = jnp.concatenate([feat, feat_a], axis=0)
    a_pad = jnp.zeros((DH, 128), jnp.float32)
    a_pad = a_pad.at[:, 0].set(att_src).at[:, 1].set(att_dst)
    H, ESED = _mm1(x2, W1, a_pad)
    es_all = ESED[:, 0]
    ed_all = ESED[:, 1]

    o, oa = _edge_attn_sc(H, es_all, ed_all, src, dst)

    h1 = jax.nn.elu(o)
    h1a = jax.nn.elu(oa)
    h2 = h1 @ W2
    h2a = h1a @ W2

    d = h2 @ Wd1 + bd1
    mu = jnp.mean(d, axis=0)
    var = jnp.var(d, axis=0)
    d = (d - mu) / jnp.sqrt(var + 1e-5) * bn_gamma + bn_beta
    d = jax.nn.elu(d)
    h3 = d @ Wd2 + bd2

    rhs = jnp.concatenate(
        [jnp.pad(h2, ((0, 0), (0, 0))), h2a], axis=1)
    P, RS = _readout_mm(graph_neigh, rhs)
    rs = RS[:, :1]
    gsum = P[:, :DOUT] / rs
    gasum = P[:, DOUT:] / rs
    gn_ = gsum / jnp.maximum(jnp.linalg.norm(gsum, axis=1, keepdims=True), 1e-12)
    gan_ = gasum / jnp.maximum(jnp.linalg.norm(gasum, axis=1, keepdims=True), 1e-12)
    g = jax.nn.sigmoid(gn_)
    ga = jax.nn.sigmoid(gan_)

    cW = g @ Wb.T
    caW = ga @ Wb.T
    ret = jnp.concatenate([
        jnp.sum(h2 * cW, axis=1, keepdims=True) + bb,
        jnp.sum(h2a * cW, axis=1, keepdims=True) + bb,
    ], axis=1)
    ret_a = jnp.concatenate([
        jnp.sum(h2a * caW, axis=1, keepdims=True) + bb,
        jnp.sum(h2 * caW, axis=1, keepdims=True) + bb,
    ], axis=1)
    return (h2, h3, ret, ret_a, h2, h2a)
